# baseline (speedup 1.0000x reference)
import jax
import jax.numpy as jnp
from jax import lax
from jax.experimental import pallas as pl
from jax.experimental.pallas import tpu as pltpu
from jax.experimental.pallas import tpu_sc as plsc

N = 10000
E = 160000
D = 128
DE = 16
NG = 64
T = 2
EPS = 1e-12

NC = 2
NS = 16
CH = 128
NCHUNK = E // CH
CPT = -(-NCHUNK // NS)
NK = D // 16

HCAP = 5120
UACC = 5128
SACC = 5128
URPT = HCAP // NS
SCHK = HCAP // 128
DUMMY = HCAP


def _lrelu(x):
    return jnp.maximum(x, 0.01 * x)


def _elu(x):
    return jnp.where(x > 0, x, jnp.exp(x) - 1.0)


_SC_MESH = dict(core_axis_name="c", subcore_axis_name="s",
                num_cores=NC, num_subcores=NS)
_SC_OUT = [
    jax.ShapeDtypeStruct((NC * HCAP, D), jnp.float32),
    jax.ShapeDtypeStruct((NC * HCAP,), jnp.float32),
]


def _sc_zero_init(wbuf, evec, u_sh, s_sh, sid):
    def zrow(i, c):
        for k in range(NK):
            wbuf[i, pl.ds(k * 16, 16)] = jnp.zeros((16,), jnp.float32)
        return c
    lax.fori_loop(0, CH, zrow, 0)
    for g in range(CH // 16):
        evec[pl.ds(g * 16, 16)] = jnp.zeros((16,), jnp.float32)
    ubase = pl.multiple_of(sid * URPT, 8)
    for r in range(URPT // CH):
        pltpu.sync_copy(wbuf, u_sh.at[pl.ds(ubase + r * CH, CH)])
    urem = URPT % CH
    if urem:
        pltpu.sync_copy(wbuf.at[pl.ds(0, urem)],
                        u_sh.at[pl.ds(ubase + (URPT // CH) * CH, urem)])
    for r in range(-(-SCHK // NS)):
        c = sid + NS * r

        @pl.when(c < SCHK)
        def _():
            off = pl.multiple_of(c * 128, 128)
            pltpu.sync_copy(evec, s_sh.at[pl.ds(off, CH)])


def _sc_writeback(u_sh, s_sh, u_out, s_out, cid, sid):
    ubase = pl.multiple_of(sid * URPT, 8)
    uout = pl.multiple_of(cid * HCAP + sid * URPT, 8)
    pltpu.sync_copy(u_sh.at[pl.ds(ubase, URPT)], u_out.at[pl.ds(uout, URPT)])
    for r in range(-(-SCHK // NS)):
        c = sid + NS * r

        @pl.when(c < SCHK)
        def _():
            off = pl.multiple_of(c * 128, 128)
            oout = pl.multiple_of(cid * HCAP + c * 128, 128)
            pltpu.sync_copy(s_sh.at[pl.ds(off, CH)], s_out.at[pl.ds(oout, CH)])


def _sc_localize(didx, didx2, cid):
    off = cid * HCAP

    def grp(g, c):
        sl = pl.ds(g * 16, 16)
        dl = didx[sl] - off
        keep = (dl >= 0) & (dl < HCAP)
        didx2[sl] = jnp.where(keep, dl, DUMMY)
        return c
    lax.fori_loop(0, CH // 16, grp, 0)


def _sc_scatter(evec, wbuf, didx2, u_sh, s_sh):
    pltpu.sync_copy(wbuf, u_sh.at[didx2], add=True)
    pltpu.sync_copy(evec, s_sh.at[didx2], add=True)


MAINJ = NCHUNK // NS
TAILC = NCHUNK - MAINJ * NS


def _sc_edge1(src, dst, nfw, s1, v, efw):
    mesh = plsc.VectorSubcoreMesh(**_SC_MESH)

    def body(src_hbm, dst_hbm, rows_hbm, s1_hbm, v_hbm, efw_hbm,
             u_out, s_out,
             sidx0, sidx1, didx0, didx1, rowbuf0, rowbuf1,
             didx2, wbuf, evec, efbuf, stab, vtab, u_sh, s_sh, gsem0, gsem1):
        cid = lax.axis_index("c")
        sid = lax.axis_index("s")
        bufs = ((sidx0, didx0, rowbuf0, gsem0),
                (sidx1, didx1, rowbuf1, gsem1))

        def issue(j, B):
            sidx, didx, rowbuf, gsem = B
            ebase = (j * NS + sid) * CH
            pltpu.sync_copy(src_hbm.at[pl.ds(ebase, CH)], sidx)
            pltpu.sync_copy(dst_hbm.at[pl.ds(ebase, CH)], didx)
            pltpu.async_copy(rows_hbm.at[sidx], rowbuf, gsem)

        def wait_in(B):
            sidx, didx, rowbuf, gsem = B
            pltpu.make_async_copy(rows_hbm.at[sidx], rowbuf, gsem).wait()

        def compute(B, j):
            sidx, didx, rowbuf, _ = B
            ebase = (j * NS + sid) * CH
            pltpu.sync_copy(efw_hbm.at[pl.ds(ebase, CH)], efbuf)
            _sc_localize(didx, didx2, cid)
            lane = lax.broadcasted_iota(jnp.int32, (16,), 0)

            def grp(g, c):
                sl = pl.ds(g * 16, 16)
                d2v = didx2[sl]
                s1v = plsc.load_gather(stab, [didx[sl]])
                accs = [jnp.zeros((16,), jnp.float32) for _ in range(16)]
                for k in range(NK):
                    ksl = pl.ds(k * 16, 16)
                    vk = vtab[ksl]
                    for t in range(16):
                        i = g * 16 + t
                        gg = rowbuf[i, ksl] + efbuf[i, ksl]
                        he = jnp.maximum(gg, 0.01 * gg)
                        rowbuf[i, ksl] = he
                        accs[t] = accs[t] + he * vk
                dotv = jnp.zeros((16,), jnp.float32)
                for t in range(16):
                    dotv = jnp.where(lane == t, jnp.sum(accs[t]), dotv)
                x = s1v + dotv
                ev = jnp.exp(jnp.maximum(x, 0.01 * x))
                evec[sl] = ev
                for t in range(16):
                    i = g * 16 + t

                    @pl.when(d2v[t] < DUMMY)
                    def _():
                        e = ev[t]
                        for k in range(NK):
                            ksl = pl.ds(k * 16, 16)
                            wbuf[i, ksl] = e * rowbuf[i, ksl]
                return c
            lax.fori_loop(0, CH // 16, grp, 0)
            pltpu.sync_copy(wbuf, u_sh.at[didx2], add=True)
            pltpu.sync_copy(evec, s_sh.at[didx2], add=True)

        pltpu.sync_copy(s1_hbm, stab)
        pltpu.sync_copy(v_hbm, vtab)
        issue(0, bufs[0])
        _sc_zero_init(wbuf, evec, u_sh, s_sh, sid)
        plsc.subcore_barrier()

        def loop(jj, c):
            for b in range(2):
                j = jj * 2 + b
                B = bufs[b]
                wait_in(B)

                @pl.when(j + 1 < MAINJ)
                def _():
                    issue(j + 1, bufs[1 - b])
                compute(B, j)
            return c
        lax.fori_loop(0, MAINJ // 2, loop, 0)

        @pl.when(sid < TAILC)
        def _():
            B = bufs[0]
            issue(MAINJ, B)
            wait_in(B)
            compute(B, MAINJ)

        plsc.subcore_barrier()
        _sc_writeback(u_sh, s_sh, u_out, s_out, cid, sid)

    f = pl.kernel(
        body,
        out_type=_SC_OUT,
        mesh=mesh,
        compiler_params=pltpu.CompilerParams(needs_layout_passes=False),
        scratch_types=(
            [pltpu.VMEM((CH,), jnp.int32)] * 4
            + [pltpu.VMEM((CH, D), jnp.float32)] * 2
            + [pltpu.VMEM((CH,), jnp.int32)]
            + [pltpu.VMEM((CH, D), jnp.float32)]
            + [pltpu.VMEM((CH,), jnp.float32)]
            + [pltpu.VMEM((CH, D), jnp.float32)]
            + [pltpu.VMEM((N,), jnp.float32), pltpu.VMEM((D,), jnp.float32),
               pltpu.VMEM_SHARED((UACC, D), jnp.float32),
               pltpu.VMEM_SHARED((SACC,), jnp.float32)]
            + [pltpu.SemaphoreType.DMA] * 2
        ),
    )
    return f(src, dst, nfw, s1, v, efw)


def _sc_edge2(src, dst, hp, t1, t2):
    mesh = plsc.VectorSubcoreMesh(**_SC_MESH)

    def body(src_hbm, dst_hbm, rows_hbm, t1_hbm, t2_hbm,
             u_out, s_out,
             sidx0, sidx1, didx0, didx1, rowbuf0, rowbuf1,
             didx2, wbuf, evec, t1tab, t2tab, u_sh, s_sh, gsem0, gsem1):
        cid = lax.axis_index("c")
        sid = lax.axis_index("s")
        bufs = ((sidx0, didx0, rowbuf0, gsem0),
                (sidx1, didx1, rowbuf1, gsem1))

        def issue(j, B):
            sidx, didx, rowbuf, gsem = B
            ebase = (j * NS + sid) * CH
            pltpu.sync_copy(src_hbm.at[pl.ds(ebase, CH)], sidx)
            pltpu.sync_copy(dst_hbm.at[pl.ds(ebase, CH)], didx)
            pltpu.async_copy(rows_hbm.at[sidx], rowbuf, gsem)

        def wait_in(B):
            sidx, didx, rowbuf, gsem = B
            pltpu.make_async_copy(rows_hbm.at[sidx], rowbuf, gsem).wait()

        def compute(B, j):
            sidx, didx, rowbuf, _ = B
            _sc_localize(didx, didx2, cid)

            def grp(g, c):
                sl = pl.ds(g * 16, 16)
                d2v = didx2[sl]
                x = (plsc.load_gather(t1tab, [didx[sl]])
                     + plsc.load_gather(t2tab, [sidx[sl]]))
                ev = jnp.exp(jnp.maximum(x, 0.01 * x))
                evec[sl] = ev
                for t in range(16):
                    i = g * 16 + t

                    @pl.when(d2v[t] < DUMMY)
                    def _():
                        e = ev[t]
                        for k in range(NK):
                            ksl = pl.ds(k * 16, 16)
                            wbuf[i, ksl] = e * rowbuf[i, ksl]
                return c
            lax.fori_loop(0, CH // 16, grp, 0)
            pltpu.sync_copy(wbuf, u_sh.at[didx2], add=True)
            pltpu.sync_copy(evec, s_sh.at[didx2], add=True)

        pltpu.sync_copy(t1_hbm, t1tab)
        pltpu.sync_copy(t2_hbm, t2tab)
        issue(0, bufs[0])
        _sc_zero_init(wbuf, evec, u_sh, s_sh, sid)
        plsc.subcore_barrier()

        def loop(jj, c):
            for b in range(2):
                j = jj * 2 + b
                B = bufs[b]
                wait_in(B)

                @pl.when(j + 1 < MAINJ)
                def _():
                    issue(j + 1, bufs[1 - b])
                compute(B, j)
            return c
        lax.fori_loop(0, MAINJ // 2, loop, 0)

        @pl.when(sid < TAILC)
        def _():
            B = bufs[0]
            issue(MAINJ, B)
            wait_in(B)
            compute(B, MAINJ)

        plsc.subcore_barrier()
        _sc_writeback(u_sh, s_sh, u_out, s_out, cid, sid)

    f = pl.kernel(
        body,
        out_type=_SC_OUT,
        mesh=mesh,
        compiler_params=pltpu.CompilerParams(needs_layout_passes=False),
        scratch_types=(
            [pltpu.VMEM((CH,), jnp.int32)] * 4
            + [pltpu.VMEM((CH, D), jnp.float32)] * 2
            + [pltpu.VMEM((CH,), jnp.int32)]
            + [pltpu.VMEM((CH, D), jnp.float32)]
            + [pltpu.VMEM((CH,), jnp.float32)]
            + [pltpu.VMEM((N,), jnp.float32), pltpu.VMEM((N,), jnp.float32),
               pltpu.VMEM_SHARED((UACC, D), jnp.float32),
               pltpu.VMEM_SHARED((SACC,), jnp.float32)]
            + [pltpu.SemaphoreType.DMA] * 2
        ),
    )
    return f(src, dst, hp, t1, t2)


def _prologue_body(nf_ref, wpn_t, bpn, wnode_t, bpe1, u, hv_ref, nfw_ref, s1_ref):
    x = nf_ref[...]
    hv = _lrelu(jnp.dot(x, wpn_t[...], preferred_element_type=jnp.float32) + bpn[...])
    hv_ref[...] = hv
    nfw_ref[...] = jnp.dot(x, wnode_t[...], preferred_element_type=jnp.float32) + bpe1[...]
    s1_ref[...] = jnp.dot(hv, u[...], preferred_element_type=jnp.float32)


def _prologue(nf, wpn_t, bpn, wnode_t, bpe1, u):
    B = 2000
    blk = lambda: pl.BlockSpec((B, D), lambda i: (i, 0))
    full = lambda r, c: pl.BlockSpec((r, c), lambda i: (0, 0))
    return pl.pallas_call(
        _prologue_body,
        grid=(N // B,),
        in_specs=[blk(), full(D, D), full(1, D), full(D, D), full(1, D), full(D, 1)],
        out_specs=[blk(), blk(), pl.BlockSpec((B, 1), lambda i: (i, 0))],
        out_shape=[
            jax.ShapeDtypeStruct((N, D), jnp.float32),
            jax.ShapeDtypeStruct((N, D), jnp.float32),
            jax.ShapeDtypeStruct((N, 1), jnp.float32),
        ],
    )(nf, wpn_t, bpn, wnode_t, bpe1, u)


def _efw_body(ef_ref, wedge_t, out_ref):
    out_ref[...] = jnp.dot(ef_ref[...], wedge_t[...],
                           preferred_element_type=jnp.float32)


def _efw(ef, wedge_t):
    B = 8000
    return pl.pallas_call(
        _efw_body,
        grid=(E // B,),
        in_specs=[pl.BlockSpec((B, DE), lambda i: (i, 0)),
                  pl.BlockSpec((DE, D), lambda i: (0, 0))],
        out_specs=pl.BlockSpec((B, D), lambda i: (i, 0)),
        out_shape=jax.ShapeDtypeStruct((E, D), jnp.float32),
    )(ef, wedge_t)


def _gru_update(x, h, wi_t, wh_t, bi, bh):
    gi = jnp.dot(x, wi_t, preferred_element_type=jnp.float32) + bi
    gh = jnp.dot(h, wh_t, preferred_element_type=jnp.float32) + bh
    r = jax.nn.sigmoid(gi[:, :D] + gh[:, :D])
    z = jax.nn.sigmoid(gi[:, D:2 * D] + gh[:, D:2 * D])
    n = jnp.tanh(gi[:, 2 * D:] + r * gh[:, 2 * D:])
    return (1.0 - z) * n + z * h


def _mid_body(u_ref, sn_ref, hv_ref, wet_t, bet, wi_t, wh_t, bi, bh,
              w12, wpn2_t, bpn2, h_ref, t12_ref, hp_ref):
    sn = sn_ref[...]
    S = u_ref[...] / (sn + EPS)
    s0 = sn / (sn + EPS)
    c = jnp.dot(S, wet_t[...], preferred_element_type=jnp.float32) + s0 * bet[...]
    hv = hv_ref[...]
    h = jnp.maximum(_gru_update(_elu(c), hv, wi_t[...], wh_t[...], bi[...], bh[...]), 0.0)
    h_ref[...] = h
    t12_ref[...] = jnp.dot(h, w12[...], preferred_element_type=jnp.float32)
    hp_ref[...] = jnp.dot(h, wpn2_t[...], preferred_element_type=jnp.float32) + bpn2[...]


def _mid(u, sn, hv, wet_t, bet, wi_t, wh_t, bi, bh, w12, wpn2_t, bpn2):
    B = 2000
    blk = pl.BlockSpec((B, D), lambda i: (i, 0))
    sblk = pl.BlockSpec((B, 1), lambda i: (i, 0))
    full = lambda r, c: pl.BlockSpec((r, c), lambda i: (0, 0))
    return pl.pallas_call(
        _mid_body,
        grid=(N // B,),
        in_specs=[blk, sblk, blk, full(D, D), full(1, D),
                  full(D, 3 * D), full(D, 3 * D), full(1, 3 * D), full(1, 3 * D),
                  full(D, 2), full(D, D), full(1, D)],
        out_specs=[blk, pl.BlockSpec((B, 2), lambda i: (i, 0)), blk],
        out_shape=[
            jax.ShapeDtypeStruct((N, D), jnp.float32),
            jax.ShapeDtypeStruct((N, 2), jnp.float32),
            jax.ShapeDtypeStruct((N, D), jnp.float32),
        ],
    )(u, sn, hv, wet_t, bet, wi_t, wh_t, bi, bh, w12, wpn2_t, bpn2)


def _post_body(u_ref, sn_ref, h_ref, wi_t, wh_t, bi, bh, wc1, bc1,
               h2_ref, q_ref):
    c2 = u_ref[...] / (sn_ref[...] + EPS)
    h = h_ref[...]
    h2 = jnp.maximum(_gru_update(_elu(c2), h, wi_t[...], wh_t[...], bi[...], bh[...]), 0.0)
    h2_ref[...] = h2
    q_ref[...] = jnp.dot(h2, wc1[...], preferred_element_type=jnp.float32) + bc1[...]


def _post(u, sn, h, wi_t, wh_t, bi, bh, wc1, bc1):
    B = 2000
    blk = pl.BlockSpec((B, D), lambda i: (i, 0))
    sblk = pl.BlockSpec((B, 1), lambda i: (i, 0))
    full = lambda r, c: pl.BlockSpec((r, c), lambda i: (0, 0))
    return pl.pallas_call(
        _post_body,
        grid=(N // B,),
        in_specs=[blk, sblk, blk,
                  full(D, 3 * D), full(D, 3 * D), full(1, 3 * D), full(1, 3 * D),
                  full(D, T), full(1, T)],
        out_specs=[blk, pl.BlockSpec((B, T), lambda i: (i, 0))],
        out_shape=[
            jax.ShapeDtypeStruct((N, D), jnp.float32),
            jax.ShapeDtypeStruct((N, T), jnp.float32),
        ],
    )(u, sn, h, wi_t, wh_t, bi, bh, wc1, bc1)


_RB = 2000
_RNB = N // _RB


def _readout_body(h2_ref, q_ref, gid_ref, wc2_ref, wpn3_t_ref, bpn3_ref,
                  wi3_t_ref, wh3_t_ref, bi3_ref, bh3_ref, wpred, bpred,
                  out_ref, gf, zu, s3, gvec):
    t = pl.program_id(0)
    j = pl.program_id(1)
    h2 = h2_ref[...]
    gidv = gid_ref[0, 0, :]
    onehot = (jax.lax.broadcasted_iota(jnp.int32, (NG, _RB), 0)
              == gidv[None, :]).astype(jnp.float32)

    @pl.when((t == 0) & (j == 0))
    def _():
        gf[...] = jnp.zeros((NG, D), jnp.float32)

    @pl.when(t == 0)
    def _():
        gf[...] += jnp.dot(onehot, h2, preferred_element_type=jnp.float32)

    @pl.when(t > 0)
    def _():
        tm = t - 1

        @pl.when(j == 0)
        def _():
            gvec[...] = jnp.dot(gf[...], wc2_ref[tm],
                                preferred_element_type=jnp.float32)
            zu[...] = jnp.zeros((NG, D), jnp.float32)
            s3[...] = jnp.zeros((NG, 1), jnp.float32)

        q = q_ref[...]
        qcol = jnp.where(tm == 0, q[:, 0:1], q[:, 1:2])
        x = qcol + jnp.dot(onehot.T, gvec[...], preferred_element_type=jnp.float32)
        e3 = jnp.exp(jnp.maximum(x, 0.01 * x))
        zu[...] += jnp.dot(onehot * e3[:, 0][None, :], h2,
                           preferred_element_type=jnp.float32)
        s3[...] += jnp.dot(onehot, e3, preferred_element_type=jnp.float32)

        @pl.when(j == _RNB - 1)
        def _():
            sv = s3[...]
            z = (jnp.dot(zu[...] / (sv + EPS), wpn3_t_ref[tm],
                         preferred_element_type=jnp.float32)
                 + (sv / (sv + EPS)) * bpn3_ref[tm])
            gfv = gf[...]
            gi = jnp.dot(_elu(z), wi3_t_ref[tm],
                         preferred_element_type=jnp.float32) + bi3_ref[tm]
            gh = jnp.dot(gfv, wh3_t_ref[tm],
                         preferred_element_type=jnp.float32) + bh3_ref[tm]
            r = jax.nn.sigmoid(gi[:, :D] + gh[:, :D])
            zz = jax.nn.sigmoid(gi[:, D:2 * D] + gh[:, D:2 * D])
            n = jnp.tanh(gi[:, 2 * D:] + r * gh[:, 2 * D:])
            gf[...] = jnp.maximum((1.0 - zz) * n + zz * gfv, 0.0)

            @pl.when(tm == T - 1)
            def _():
                out_ref[...] = jnp.dot(gf[...], wpred[...],
                                       preferred_element_type=jnp.float32) + bpred[...]


def _readout(h2, q, gid3, wc2, wpn3_t, bpn3, wi3_t, wh3_t, bi3, bh3, wpred, bpred):
    blk = pl.BlockSpec((_RB, D), lambda t, j: (j, 0))
    full = lambda *s: pl.BlockSpec(s, lambda t, j: (0,) * len(s))
    return pl.pallas_call(
        _readout_body,
        grid=(T + 1, _RNB),
        in_specs=[blk, pl.BlockSpec((_RB, T), lambda t, j: (j, 0)),
                  pl.BlockSpec((1, 1, _RB), lambda t, j: (j, 0, 0)),
                  full(T, D, 1), full(T, D, D), full(T, 1, D),
                  full(T, D, 3 * D), full(T, D, 3 * D),
                  full(T, 1, 3 * D), full(T, 1, 3 * D),
                  full(D, 1), full(1, 1)],
        out_specs=pl.BlockSpec((NG, 1), lambda t, j: (0, 0)),
        out_shape=jax.ShapeDtypeStruct((NG, 1), jnp.float32),
        scratch_shapes=[
            pltpu.VMEM((NG, D), jnp.float32),
            pltpu.VMEM((NG, D), jnp.float32),
            pltpu.VMEM((NG, 1), jnp.float32),
            pltpu.VMEM((NG, 1), jnp.float32),
        ],
        compiler_params=pltpu.CompilerParams(
            dimension_semantics=("arbitrary", "arbitrary")),
    )(h2, q, gid3, wc2, wpn3_t, bpn3, wi3_t, wh3_t, bi3, bh3, wpred, bpred)


def kernel(node_feats, edge_feats, params, edge_index, node_graph_ids):
    p = params
    src = edge_index[0].astype(jnp.int32)
    dst = edge_index[1].astype(jnp.int32)
    gid3 = node_graph_ids.astype(jnp.int32).reshape(_RNB, 1, _RB)

    wpn_t = p['W_pn'].T
    wnode_t = p['W_pe1'][:, :D].T
    wedge_t = p['W_pe1'][:, D:].T
    u = (p['W_pe2'][0, :D]).reshape(D, 1)
    v = p['W_pe2'][0, D:]
    s1b = p['b_pe2'][0]

    hv, nfw, s1 = _prologue(node_feats, wpn_t, p['b_pn'].reshape(1, D),
                            wnode_t, p['b_pe1'].reshape(1, D), u)
    efw = _efw(edge_feats, wedge_t)

    u1, s1o = _sc_edge1(src, dst, nfw, s1[:, 0] + s1b, v, efw)
    sn1 = s1o.reshape(NC * HCAP, 1)

    w12 = jnp.stack([p['W_e'][0, :D], p['W_e'][0, D:]], axis=1)
    h, t12, hp = _mid(
        u1, sn1, hv, p['W_et'].T, p['b_et'].reshape(1, D),
        p['Wi1'].T, p['Wh1'].T, p['bi1'].reshape(1, 3 * D), p['bh1'].reshape(1, 3 * D),
        w12, p['W_pn2'].T, p['b_pn2'].reshape(1, D))

    t1 = t12[:, 0] + p['b_e'][0]
    t2 = t12[:, 1]
    u2, s2o = _sc_edge2(src, dst, hp, t1, t2)
    sn2 = s2o.reshape(NC * HCAP, 1)

    wc1 = jnp.stack([p['W_cl'][0, 0, :D], p['W_cl'][1, 0, :D]], axis=1)
    bc1 = p['b_cl'][:, 0].reshape(1, T)
    h2, q = _post(u2, sn2, h, p['Wi2'].T, p['Wh2'].T,
                  p['bi2'].reshape(1, 3 * D), p['bh2'].reshape(1, 3 * D), wc1, bc1)

    wc2 = p['W_cl'][:, 0, D:].reshape(T, D, 1)
    wpn3_t = jnp.transpose(p['W_pn3'], (0, 2, 1))
    bpn3 = p['b_pn3'].reshape(T, 1, D)
    wi3_t = jnp.transpose(p['Wi3'], (0, 2, 1))
    wh3_t = jnp.transpose(p['Wh3'], (0, 2, 1))
    bi3 = p['bi3'].reshape(T, 1, 3 * D)
    bh3 = p['bh3'].reshape(T, 1, 3 * D)

    out = _readout(h2, q, gid3, wc2, wpn3_t, bpn3, wi3_t, wh3_t, bi3, bh3,
                   p['W_pred'].T, p['b_pred'].reshape(1, 1))
    return out


# speedup vs baseline: 1.0580x; 1.0580x over previous
import jax
import jax.numpy as jnp
from jax import lax
from jax.experimental import pallas as pl
from jax.experimental.pallas import tpu as pltpu
from jax.experimental.pallas import tpu_sc as plsc

N = 10000
E = 160000
D = 128
DE = 16
NG = 64
T = 2
EPS = 1e-12

NC = 2
NS = 16
CH = 128
NCHUNK = E // CH
CPT = -(-NCHUNK // NS)
NK = D // 16

HCAP = 5120
UACC = 5128
SACC = 5128
URPT = HCAP // NS
SCHK = HCAP // 128
DUMMY = HCAP


def _lrelu(x):
    return jnp.maximum(x, 0.01 * x)


def _elu(x):
    return jnp.where(x > 0, x, jnp.exp(x) - 1.0)


_SC_MESH = dict(core_axis_name="c", subcore_axis_name="s",
                num_cores=NC, num_subcores=NS)
_SC_OUT = [
    jax.ShapeDtypeStruct((NC * HCAP, D), jnp.float32),
    jax.ShapeDtypeStruct((NC * HCAP,), jnp.float32),
]


def _sc_zero_init(wbuf, evec, u_sh, s_sh, sid):
    def zrow(i, c):
        for k in range(NK):
            wbuf[i, pl.ds(k * 16, 16)] = jnp.zeros((16,), jnp.float32)
        return c
    lax.fori_loop(0, CH, zrow, 0)
    for g in range(CH // 16):
        evec[pl.ds(g * 16, 16)] = jnp.zeros((16,), jnp.float32)
    ubase = pl.multiple_of(sid * URPT, 8)
    for r in range(URPT // CH):
        pltpu.sync_copy(wbuf, u_sh.at[pl.ds(ubase + r * CH, CH)])
    urem = URPT % CH
    if urem:
        pltpu.sync_copy(wbuf.at[pl.ds(0, urem)],
                        u_sh.at[pl.ds(ubase + (URPT // CH) * CH, urem)])
    for r in range(-(-SCHK // NS)):
        c = sid + NS * r

        @pl.when(c < SCHK)
        def _():
            off = pl.multiple_of(c * 128, 128)
            pltpu.sync_copy(evec, s_sh.at[pl.ds(off, CH)])


def _sc_writeback(u_sh, s_sh, u_out, s_out, cid, sid):
    ubase = pl.multiple_of(sid * URPT, 8)
    uout = pl.multiple_of(cid * HCAP + sid * URPT, 8)
    pltpu.sync_copy(u_sh.at[pl.ds(ubase, URPT)], u_out.at[pl.ds(uout, URPT)])
    for r in range(-(-SCHK // NS)):
        c = sid + NS * r

        @pl.when(c < SCHK)
        def _():
            off = pl.multiple_of(c * 128, 128)
            oout = pl.multiple_of(cid * HCAP + c * 128, 128)
            pltpu.sync_copy(s_sh.at[pl.ds(off, CH)], s_out.at[pl.ds(oout, CH)])


def _sc_localize(didx, didx2, cid):
    off = cid * HCAP

    def grp(g, c):
        sl = pl.ds(g * 16, 16)
        dl = didx[sl] - off
        keep = (dl >= 0) & (dl < HCAP)
        didx2[sl] = jnp.where(keep, dl, DUMMY)
        return c
    lax.fori_loop(0, CH // 16, grp, 0)


def _sc_scatter(evec, wbuf, didx2, u_sh, s_sh):
    pltpu.sync_copy(wbuf, u_sh.at[didx2], add=True)
    pltpu.sync_copy(evec, s_sh.at[didx2], add=True)


MAINJ = NCHUNK // NS
TAILC = NCHUNK - MAINJ * NS


def _sc_edge1(src, dst, nfw, s1, v, efw):
    mesh = plsc.VectorSubcoreMesh(**_SC_MESH)

    def body(src_hbm, dst_hbm, rows_hbm, s1_hbm, v_hbm, efw_hbm,
             u_out, s_out,
             sidx0, sidx1, didx0, didx1, rowbuf0, rowbuf1,
             didx2, wbuf, evec, efbuf, stab, vtab, u_sh, s_sh, gsem0, gsem1):
        cid = lax.axis_index("c")
        sid = lax.axis_index("s")
        bufs = ((sidx0, didx0, rowbuf0, gsem0),
                (sidx1, didx1, rowbuf1, gsem1))

        def issue(j, B):
            sidx, didx, rowbuf, gsem = B
            ebase = (j * NS + sid) * CH
            pltpu.sync_copy(src_hbm.at[pl.ds(ebase, CH)], sidx)
            pltpu.sync_copy(dst_hbm.at[pl.ds(ebase, CH)], didx)
            pltpu.async_copy(rows_hbm.at[sidx], rowbuf, gsem)

        def wait_in(B):
            sidx, didx, rowbuf, gsem = B
            pltpu.make_async_copy(rows_hbm.at[sidx], rowbuf, gsem).wait()

        def compute(B, j):
            sidx, didx, rowbuf, _ = B
            ebase = (j * NS + sid) * CH
            pltpu.sync_copy(efw_hbm.at[pl.ds(ebase, CH)], efbuf)
            _sc_localize(didx, didx2, cid)
            lane = lax.broadcasted_iota(jnp.int32, (16,), 0)

            def grp(g, c):
                sl = pl.ds(g * 16, 16)
                d2v = didx2[sl]
                s1v = plsc.load_gather(stab, [didx[sl]])
                vks = [vtab[pl.ds(k * 16, 16)] for k in range(NK)]
                dotv = jnp.zeros((16,), jnp.float32)
                for t in range(16):
                    i = g * 16 + t

                    def _dot():
                        acc0 = jnp.zeros((16,), jnp.float32)
                        acc1 = jnp.zeros((16,), jnp.float32)
                        for k in range(NK):
                            ksl = pl.ds(k * 16, 16)
                            gg = rowbuf[i, ksl] + efbuf[i, ksl]
                            he = jnp.maximum(gg, 0.01 * gg)
                            rowbuf[i, ksl] = he
                            if k % 2 == 0:
                                acc0 = acc0 + he * vks[k]
                            else:
                                acc1 = acc1 + he * vks[k]
                        return jnp.sum(acc0 + acc1)

                    d = lax.cond(d2v[t] < DUMMY, _dot,
                                 lambda: jnp.float32(0.0))
                    dotv = jnp.where(lane == t, d, dotv)
                x = s1v + dotv
                ev = jnp.exp(jnp.maximum(x, 0.01 * x))
                evec[sl] = ev
                for t in range(16):
                    i = g * 16 + t

                    @pl.when(d2v[t] < DUMMY)
                    def _():
                        e = ev[t]
                        for k in range(NK):
                            ksl = pl.ds(k * 16, 16)
                            wbuf[i, ksl] = e * rowbuf[i, ksl]
                return c
            lax.fori_loop(0, CH // 16, grp, 0)
            pltpu.sync_copy(wbuf, u_sh.at[didx2], add=True)
            pltpu.sync_copy(evec, s_sh.at[didx2], add=True)

        pltpu.sync_copy(s1_hbm, stab)
        pltpu.sync_copy(v_hbm, vtab)
        issue(0, bufs[0])
        _sc_zero_init(wbuf, evec, u_sh, s_sh, sid)
        plsc.subcore_barrier()

        def loop(jj, c):
            for b in range(2):
                j = jj * 2 + b
                B = bufs[b]
                wait_in(B)

                @pl.when(j + 1 < MAINJ)
                def _():
                    issue(j + 1, bufs[1 - b])
                compute(B, j)
            return c
        lax.fori_loop(0, MAINJ // 2, loop, 0)

        @pl.when(sid < TAILC)
        def _():
            B = bufs[0]
            issue(MAINJ, B)
            wait_in(B)
            compute(B, MAINJ)

        plsc.subcore_barrier()
        _sc_writeback(u_sh, s_sh, u_out, s_out, cid, sid)

    f = pl.kernel(
        body,
        out_type=_SC_OUT,
        mesh=mesh,
        compiler_params=pltpu.CompilerParams(needs_layout_passes=False),
        scratch_types=(
            [pltpu.VMEM((CH,), jnp.int32)] * 4
            + [pltpu.VMEM((CH, D), jnp.float32)] * 2
            + [pltpu.VMEM((CH,), jnp.int32)]
            + [pltpu.VMEM((CH, D), jnp.float32)]
            + [pltpu.VMEM((CH,), jnp.float32)]
            + [pltpu.VMEM((CH, D), jnp.float32)]
            + [pltpu.VMEM((N,), jnp.float32), pltpu.VMEM((D,), jnp.float32),
               pltpu.VMEM_SHARED((UACC, D), jnp.float32),
               pltpu.VMEM_SHARED((SACC,), jnp.float32)]
            + [pltpu.SemaphoreType.DMA] * 2
        ),
    )
    return f(src, dst, nfw, s1, v, efw)


def _sc_edge2(src, dst, hp, t1, t2):
    mesh = plsc.VectorSubcoreMesh(**_SC_MESH)

    def body(src_hbm, dst_hbm, rows_hbm, t1_hbm, t2_hbm,
             u_out, s_out,
             sidx0, sidx1, didx0, didx1, rowbuf0, rowbuf1,
             didx2, wbuf, evec, t1tab, t2tab, u_sh, s_sh, gsem0, gsem1):
        cid = lax.axis_index("c")
        sid = lax.axis_index("s")
        bufs = ((sidx0, didx0, rowbuf0, gsem0),
                (sidx1, didx1, rowbuf1, gsem1))

        def issue(j, B):
            sidx, didx, rowbuf, gsem = B
            ebase = (j * NS + sid) * CH
            pltpu.sync_copy(src_hbm.at[pl.ds(ebase, CH)], sidx)
            pltpu.sync_copy(dst_hbm.at[pl.ds(ebase, CH)], didx)
            pltpu.async_copy(rows_hbm.at[sidx], rowbuf, gsem)

        def wait_in(B):
            sidx, didx, rowbuf, gsem = B
            pltpu.make_async_copy(rows_hbm.at[sidx], rowbuf, gsem).wait()

        def compute(B, j):
            sidx, didx, rowbuf, _ = B
            _sc_localize(didx, didx2, cid)

            def grp(g, c):
                sl = pl.ds(g * 16, 16)
                d2v = didx2[sl]
                x = (plsc.load_gather(t1tab, [didx[sl]])
                     + plsc.load_gather(t2tab, [sidx[sl]]))
                ev = jnp.exp(jnp.maximum(x, 0.01 * x))
                evec[sl] = ev
                for t in range(16):
                    i = g * 16 + t

                    @pl.when(d2v[t] < DUMMY)
                    def _():
                        e = ev[t]
                        for k in range(NK):
                            ksl = pl.ds(k * 16, 16)
                            wbuf[i, ksl] = e * rowbuf[i, ksl]
                return c
            lax.fori_loop(0, CH // 16, grp, 0)
            pltpu.sync_copy(wbuf, u_sh.at[didx2], add=True)
            pltpu.sync_copy(evec, s_sh.at[didx2], add=True)

        pltpu.sync_copy(t1_hbm, t1tab)
        pltpu.sync_copy(t2_hbm, t2tab)
        issue(0, bufs[0])
        _sc_zero_init(wbuf, evec, u_sh, s_sh, sid)
        plsc.subcore_barrier()

        def loop(jj, c):
            for b in range(2):
                j = jj * 2 + b
                B = bufs[b]
                wait_in(B)

                @pl.when(j + 1 < MAINJ)
                def _():
                    issue(j + 1, bufs[1 - b])
                compute(B, j)
            return c
        lax.fori_loop(0, MAINJ // 2, loop, 0)

        @pl.when(sid < TAILC)
        def _():
            B = bufs[0]
            issue(MAINJ, B)
            wait_in(B)
            compute(B, MAINJ)

        plsc.subcore_barrier()
        _sc_writeback(u_sh, s_sh, u_out, s_out, cid, sid)

    f = pl.kernel(
        body,
        out_type=_SC_OUT,
        mesh=mesh,
        compiler_params=pltpu.CompilerParams(needs_layout_passes=False),
        scratch_types=(
            [pltpu.VMEM((CH,), jnp.int32)] * 4
            + [pltpu.VMEM((CH, D), jnp.float32)] * 2
            + [pltpu.VMEM((CH,), jnp.int32)]
            + [pltpu.VMEM((CH, D), jnp.float32)]
            + [pltpu.VMEM((CH,), jnp.float32)]
            + [pltpu.VMEM((N,), jnp.float32), pltpu.VMEM((N,), jnp.float32),
               pltpu.VMEM_SHARED((UACC, D), jnp.float32),
               pltpu.VMEM_SHARED((SACC,), jnp.float32)]
            + [pltpu.SemaphoreType.DMA] * 2
        ),
    )
    return f(src, dst, hp, t1, t2)


def _prologue_body(nf_ref, wpn_t, bpn, wnode_t, bpe1, u, hv_ref, nfw_ref, s1_ref):
    x = nf_ref[...]
    hv = _lrelu(jnp.dot(x, wpn_t[...], preferred_element_type=jnp.float32) + bpn[...])
    hv_ref[...] = hv
    nfw_ref[...] = jnp.dot(x, wnode_t[...], preferred_element_type=jnp.float32) + bpe1[...]
    s1_ref[...] = jnp.dot(hv, u[...], preferred_element_type=jnp.float32)


def _prologue(nf, wpn_t, bpn, wnode_t, bpe1, u):
    B = 2000
    blk = lambda: pl.BlockSpec((B, D), lambda i: (i, 0))
    full = lambda r, c: pl.BlockSpec((r, c), lambda i: (0, 0))
    return pl.pallas_call(
        _prologue_body,
        grid=(N // B,),
        in_specs=[blk(), full(D, D), full(1, D), full(D, D), full(1, D), full(D, 1)],
        out_specs=[blk(), blk(), pl.BlockSpec((B, 1), lambda i: (i, 0))],
        out_shape=[
            jax.ShapeDtypeStruct((N, D), jnp.float32),
            jax.ShapeDtypeStruct((N, D), jnp.float32),
            jax.ShapeDtypeStruct((N, 1), jnp.float32),
        ],
    )(nf, wpn_t, bpn, wnode_t, bpe1, u)


def _efw_body(ef_ref, wedge_t, out_ref):
    out_ref[...] = jnp.dot(ef_ref[...], wedge_t[...],
                           preferred_element_type=jnp.float32)


def _efw(ef, wedge_t):
    B = 8000
    return pl.pallas_call(
        _efw_body,
        grid=(E // B,),
        in_specs=[pl.BlockSpec((B, DE), lambda i: (i, 0)),
                  pl.BlockSpec((DE, D), lambda i: (0, 0))],
        out_specs=pl.BlockSpec((B, D), lambda i: (i, 0)),
        out_shape=jax.ShapeDtypeStruct((E, D), jnp.float32),
    )(ef, wedge_t)


def _gru_update(x, h, wi_t, wh_t, bi, bh):
    gi = jnp.dot(x, wi_t, preferred_element_type=jnp.float32) + bi
    gh = jnp.dot(h, wh_t, preferred_element_type=jnp.float32) + bh
    r = jax.nn.sigmoid(gi[:, :D] + gh[:, :D])
    z = jax.nn.sigmoid(gi[:, D:2 * D] + gh[:, D:2 * D])
    n = jnp.tanh(gi[:, 2 * D:] + r * gh[:, 2 * D:])
    return (1.0 - z) * n + z * h


def _mid_body(u_ref, sn_ref, hv_ref, wet_t, bet, wi_t, wh_t, bi, bh,
              w12, wpn2_t, bpn2, h_ref, t12_ref, hp_ref):
    sn = sn_ref[...]
    S = u_ref[...] / (sn + EPS)
    s0 = sn / (sn + EPS)
    c = jnp.dot(S, wet_t[...], preferred_element_type=jnp.float32) + s0 * bet[...]
    hv = hv_ref[...]
    h = jnp.maximum(_gru_update(_elu(c), hv, wi_t[...], wh_t[...], bi[...], bh[...]), 0.0)
    h_ref[...] = h
    t12_ref[...] = jnp.dot(h, w12[...], preferred_element_type=jnp.float32)
    hp_ref[...] = jnp.dot(h, wpn2_t[...], preferred_element_type=jnp.float32) + bpn2[...]


def _mid(u, sn, hv, wet_t, bet, wi_t, wh_t, bi, bh, w12, wpn2_t, bpn2):
    B = 2000
    blk = pl.BlockSpec((B, D), lambda i: (i, 0))
    sblk = pl.BlockSpec((B, 1), lambda i: (i, 0))
    full = lambda r, c: pl.BlockSpec((r, c), lambda i: (0, 0))
    return pl.pallas_call(
        _mid_body,
        grid=(N // B,),
        in_specs=[blk, sblk, blk, full(D, D), full(1, D),
                  full(D, 3 * D), full(D, 3 * D), full(1, 3 * D), full(1, 3 * D),
                  full(D, 2), full(D, D), full(1, D)],
        out_specs=[blk, pl.BlockSpec((B, 2), lambda i: (i, 0)), blk],
        out_shape=[
            jax.ShapeDtypeStruct((N, D), jnp.float32),
            jax.ShapeDtypeStruct((N, 2), jnp.float32),
            jax.ShapeDtypeStruct((N, D), jnp.float32),
        ],
    )(u, sn, hv, wet_t, bet, wi_t, wh_t, bi, bh, w12, wpn2_t, bpn2)


def _post_body(u_ref, sn_ref, h_ref, wi_t, wh_t, bi, bh, wc1, bc1,
               h2_ref, q_ref):
    c2 = u_ref[...] / (sn_ref[...] + EPS)
    h = h_ref[...]
    h2 = jnp.maximum(_gru_update(_elu(c2), h, wi_t[...], wh_t[...], bi[...], bh[...]), 0.0)
    h2_ref[...] = h2
    q_ref[...] = jnp.dot(h2, wc1[...], preferred_element_type=jnp.float32) + bc1[...]


def _post(u, sn, h, wi_t, wh_t, bi, bh, wc1, bc1):
    B = 2000
    blk = pl.BlockSpec((B, D), lambda i: (i, 0))
    sblk = pl.BlockSpec((B, 1), lambda i: (i, 0))
    full = lambda r, c: pl.BlockSpec((r, c), lambda i: (0, 0))
    return pl.pallas_call(
        _post_body,
        grid=(N // B,),
        in_specs=[blk, sblk, blk,
                  full(D, 3 * D), full(D, 3 * D), full(1, 3 * D), full(1, 3 * D),
                  full(D, T), full(1, T)],
        out_specs=[blk, pl.BlockSpec((B, T), lambda i: (i, 0))],
        out_shape=[
            jax.ShapeDtypeStruct((N, D), jnp.float32),
            jax.ShapeDtypeStruct((N, T), jnp.float32),
        ],
    )(u, sn, h, wi_t, wh_t, bi, bh, wc1, bc1)


_RB = 2000
_RNB = N // _RB


def _readout_body(h2_ref, q_ref, gid_ref, wc2_ref, wpn3_t_ref, bpn3_ref,
                  wi3_t_ref, wh3_t_ref, bi3_ref, bh3_ref, wpred, bpred,
                  out_ref, gf, zu, s3, gvec):
    t = pl.program_id(0)
    j = pl.program_id(1)
    h2 = h2_ref[...]
    gidv = gid_ref[0, 0, :]
    onehot = (jax.lax.broadcasted_iota(jnp.int32, (NG, _RB), 0)
              == gidv[None, :]).astype(jnp.float32)

    @pl.when((t == 0) & (j == 0))
    def _():
        gf[...] = jnp.zeros((NG, D), jnp.float32)

    @pl.when(t == 0)
    def _():
        gf[...] += jnp.dot(onehot, h2, preferred_element_type=jnp.float32)

    @pl.when(t > 0)
    def _():
        tm = t - 1

        @pl.when(j == 0)
        def _():
            gvec[...] = jnp.dot(gf[...], wc2_ref[tm],
                                preferred_element_type=jnp.float32)
            zu[...] = jnp.zeros((NG, D), jnp.float32)
            s3[...] = jnp.zeros((NG, 1), jnp.float32)

        q = q_ref[...]
        qcol = jnp.where(tm == 0, q[:, 0:1], q[:, 1:2])
        x = qcol + jnp.dot(onehot.T, gvec[...], preferred_element_type=jnp.float32)
        e3 = jnp.exp(jnp.maximum(x, 0.01 * x))
        zu[...] += jnp.dot(onehot * e3[:, 0][None, :], h2,
                           preferred_element_type=jnp.float32)
        s3[...] += jnp.dot(onehot, e3, preferred_element_type=jnp.float32)

        @pl.when(j == _RNB - 1)
        def _():
            sv = s3[...]
            z = (jnp.dot(zu[...] / (sv + EPS), wpn3_t_ref[tm],
                         preferred_element_type=jnp.float32)
                 + (sv / (sv + EPS)) * bpn3_ref[tm])
            gfv = gf[...]
            gi = jnp.dot(_elu(z), wi3_t_ref[tm],
                         preferred_element_type=jnp.float32) + bi3_ref[tm]
            gh = jnp.dot(gfv, wh3_t_ref[tm],
                         preferred_element_type=jnp.float32) + bh3_ref[tm]
            r = jax.nn.sigmoid(gi[:, :D] + gh[:, :D])
            zz = jax.nn.sigmoid(gi[:, D:2 * D] + gh[:, D:2 * D])
            n = jnp.tanh(gi[:, 2 * D:] + r * gh[:, 2 * D:])
            gf[...] = jnp.maximum((1.0 - zz) * n + zz * gfv, 0.0)

            @pl.when(tm == T - 1)
            def _():
                out_ref[...] = jnp.dot(gf[...], wpred[...],
                                       preferred_element_type=jnp.float32) + bpred[...]


def _readout(h2, q, gid3, wc2, wpn3_t, bpn3, wi3_t, wh3_t, bi3, bh3, wpred, bpred):
    blk = pl.BlockSpec((_RB, D), lambda t, j: (j, 0))
    full = lambda *s: pl.BlockSpec(s, lambda t, j: (0,) * len(s))
    return pl.pallas_call(
        _readout_body,
        grid=(T + 1, _RNB),
        in_specs=[blk, pl.BlockSpec((_RB, T), lambda t, j: (j, 0)),
                  pl.BlockSpec((1, 1, _RB), lambda t, j: (j, 0, 0)),
                  full(T, D, 1), full(T, D, D), full(T, 1, D),
                  full(T, D, 3 * D), full(T, D, 3 * D),
                  full(T, 1, 3 * D), full(T, 1, 3 * D),
                  full(D, 1), full(1, 1)],
        out_specs=pl.BlockSpec((NG, 1), lambda t, j: (0, 0)),
        out_shape=jax.ShapeDtypeStruct((NG, 1), jnp.float32),
        scratch_shapes=[
            pltpu.VMEM((NG, D), jnp.float32),
            pltpu.VMEM((NG, D), jnp.float32),
            pltpu.VMEM((NG, 1), jnp.float32),
            pltpu.VMEM((NG, 1), jnp.float32),
        ],
        compiler_params=pltpu.CompilerParams(
            dimension_semantics=("arbitrary", "arbitrary")),
    )(h2, q, gid3, wc2, wpn3_t, bpn3, wi3_t, wh3_t, bi3, bh3, wpred, bpred)


def kernel(node_feats, edge_feats, params, edge_index, node_graph_ids):
    p = params
    src = edge_index[0].astype(jnp.int32)
    dst = edge_index[1].astype(jnp.int32)
    gid3 = node_graph_ids.astype(jnp.int32).reshape(_RNB, 1, _RB)

    wpn_t = p['W_pn'].T
    wnode_t = p['W_pe1'][:, :D].T
    wedge_t = p['W_pe1'][:, D:].T
    u = (p['W_pe2'][0, :D]).reshape(D, 1)
    v = p['W_pe2'][0, D:]
    s1b = p['b_pe2'][0]

    hv, nfw, s1 = _prologue(node_feats, wpn_t, p['b_pn'].reshape(1, D),
                            wnode_t, p['b_pe1'].reshape(1, D), u)
    efw = _efw(edge_feats, wedge_t)

    u1, s1o = _sc_edge1(src, dst, nfw, s1[:, 0] + s1b, v, efw)
    sn1 = s1o.reshape(NC * HCAP, 1)

    w12 = jnp.stack([p['W_e'][0, :D], p['W_e'][0, D:]], axis=1)
    h, t12, hp = _mid(
        u1, sn1, hv, p['W_et'].T, p['b_et'].reshape(1, D),
        p['Wi1'].T, p['Wh1'].T, p['bi1'].reshape(1, 3 * D), p['bh1'].reshape(1, 3 * D),
        w12, p['W_pn2'].T, p['b_pn2'].reshape(1, D))

    t1 = t12[:, 0] + p['b_e'][0]
    t2 = t12[:, 1]
    u2, s2o = _sc_edge2(src, dst, hp, t1, t2)
    sn2 = s2o.reshape(NC * HCAP, 1)

    wc1 = jnp.stack([p['W_cl'][0, 0, :D], p['W_cl'][1, 0, :D]], axis=1)
    bc1 = p['b_cl'][:, 0].reshape(1, T)
    h2, q = _post(u2, sn2, h, p['Wi2'].T, p['Wh2'].T,
                  p['bi2'].reshape(1, 3 * D), p['bh2'].reshape(1, 3 * D), wc1, bc1)

    wc2 = p['W_cl'][:, 0, D:].reshape(T, D, 1)
    wpn3_t = jnp.transpose(p['W_pn3'], (0, 2, 1))
    bpn3 = p['b_pn3'].reshape(T, 1, D)
    wi3_t = jnp.transpose(p['Wi3'], (0, 2, 1))
    wh3_t = jnp.transpose(p['Wh3'], (0, 2, 1))
    bi3 = p['bi3'].reshape(T, 1, 3 * D)
    bh3 = p['bh3'].reshape(T, 1, 3 * D)

    out = _readout(h2, q, gid3, wc2, wpn3_t, bpn3, wi3_t, wh3_t, bi3, bh3,
                   p['W_pred'].T, p['b_pred'].reshape(1, 1))
    return out
